# baseline (speedup 1.0000x reference)
import functools

import jax
import jax.numpy as jnp
import numpy as np
from jax import lax
from jax.experimental import pallas as pl
from jax.experimental.pallas import tpu as pltpu
from jax.experimental.pallas import tpu_sc as plsc

NW = 32
NC = 2
NS = 16
CH = 128
CHG = 64
_CORE0_FRAC_AB = 50
_CORE0_FRAC_CD = 50


def _gather_pipeline(cpt, start_gather, wait_gather, start_wb, wait_wb):
    m = cpt // 4
    start_gather(0, 0)
    start_gather(1, 1)
    for k in range(4):
        wait_gather()
        start_wb(k, k % 4)
        if k >= 2:
            wait_wb()
        start_gather(k + 2, (k + 2) % 4)

    def body(kk, carry):
        k0 = 4 * kk
        for b in range(4):
            wait_gather()
            start_wb(k0 + b, b)
            wait_wb()
            start_gather(k0 + b + 2, (b + 2) % 4)
        return carry

    lax.fori_loop(1, m - 1, body, None)
    for k in range(cpt - 4, cpt):
        wait_gather()
        start_wb(k, k % 4)
        wait_wb()
        if k + 2 <= cpt - 1:
            start_gather(k + 2, (k + 2) % 4)
    wait_wb()
    wait_wb()


def _node_pre(h, w_hc):
    n, d = h.shape
    blk = 1000

    def body(h_ref, w_ref, a_ref, b_ref):
        ab = h_ref[...] @ w_ref[...]
        a_ref[...] = ab[:, :d]
        b_ref[...] = ab[:, d:]

    return pl.pallas_call(
        body,
        grid=(n // blk,),
        in_specs=[
            pl.BlockSpec((blk, d), lambda i: (i, 0)),
            pl.BlockSpec((d, 2 * d), lambda i: (0, 0)),
        ],
        out_specs=[
            pl.BlockSpec((blk, d), lambda i: (i, 0)),
            pl.BlockSpec((blk, d), lambda i: (i, 0)),
        ],
        out_shape=[
            jax.ShapeDtypeStruct((n, d), jnp.float32),
            jax.ShapeDtypeStruct((n, d), jnp.float32),
        ],
    )(h, w_hc)


def _sc_gather(a_tab, b_tab, row_g, col_g, ep):
    d = a_tab.shape[1]
    nch = ep // CHG
    per_pair = nch // NS
    cpt0 = _CORE0_FRAC_AB * per_pair // 100 // 4 * 4
    cpt1 = per_pair - cpt0
    mesh = plsc.VectorSubcoreMesh(core_axis_name="c", subcore_axis_name="s")

    @functools.partial(
        pl.kernel,
        out_type=[
            jax.ShapeDtypeStruct((ep, d), jnp.float32),
            jax.ShapeDtypeStruct((ep, d), jnp.float32),
        ],
        mesh=mesh,
        scratch_types=[
            pltpu.VMEM((cpt0, CHG), jnp.int32),
            pltpu.VMEM((cpt0, CHG), jnp.int32),
            pltpu.VMEM((4, CHG, d), jnp.float32),
            pltpu.VMEM((4, CHG, d), jnp.float32),
            pltpu.SemaphoreType.DMA,
            pltpu.SemaphoreType.DMA,
            pltpu.SemaphoreType.DMA,
        ],
    )
    def k(a_hbm, b_hbm, row_hbm, col_hbm,
          ar_out, bc_out,
          rowi, coli, arv, bcv, sem_g, sem_w, sem_i):
        c = lax.axis_index("c")
        s = lax.axis_index("s")

        def start_gather(kd, slot):
            pltpu.async_copy(a_hbm.at[rowi.at[kd]], arv.at[slot], sem_g)
            pltpu.async_copy(b_hbm.at[coli.at[kd]], bcv.at[slot], sem_g)

        def wait_gather():
            pltpu.make_async_copy(a_hbm.at[pl.ds(0, CHG)], arv.at[0], sem_g).wait()
            pltpu.make_async_copy(b_hbm.at[pl.ds(0, CHG)], bcv.at[0], sem_g).wait()

        def run(start_chunk, cptw):
            base0 = start_chunk * CHG
            pltpu.async_copy(row_hbm.at[pl.ds(start_chunk, cptw)],
                             rowi.at[pl.ds(0, cptw)], sem_i)
            pltpu.async_copy(col_hbm.at[pl.ds(start_chunk, cptw)],
                             coli.at[pl.ds(0, cptw)], sem_i)
            pltpu.make_async_copy(row_hbm.at[pl.ds(0, cptw)],
                                  rowi.at[pl.ds(0, cptw)], sem_i).wait()
            pltpu.make_async_copy(col_hbm.at[pl.ds(0, cptw)],
                                  coli.at[pl.ds(0, cptw)], sem_i).wait()

            def start_wb(kd, slot):
                pltpu.async_copy(arv.at[slot],
                                 ar_out.at[pl.ds(base0 + kd * CHG, CHG)], sem_w)
                pltpu.async_copy(bcv.at[slot],
                                 bc_out.at[pl.ds(base0 + kd * CHG, CHG)], sem_w)

            def wait_wb():
                pltpu.make_async_copy(a_hbm.at[pl.ds(0, CHG)], arv.at[0],
                                      sem_w).wait()
                pltpu.make_async_copy(b_hbm.at[pl.ds(0, CHG)], bcv.at[0],
                                      sem_w).wait()

            _gather_pipeline(cptw, start_gather, wait_gather, start_wb, wait_wb)

        @pl.when(c == 0)
        def _():
            run(s * cpt0, cpt0)

        @pl.when(c == 1)
        def _():
            run(NS * cpt0 + s * cpt1, cpt1)

    return k(a_tab, b_tab, row_g, col_g)


def _sc_gather_coord(coord16, row_g, col_g, ep):
    nch = ep // CHG
    per_pair = nch // NS
    cpt0 = _CORE0_FRAC_CD * per_pair // 100 // 4 * 4
    cpt1 = per_pair - cpt0
    mesh = plsc.VectorSubcoreMesh(core_axis_name="c", subcore_axis_name="s")

    @functools.partial(
        pl.kernel,
        out_type=jax.ShapeDtypeStruct((ep // 8, 128), jnp.float32),
        mesh=mesh,
        scratch_types=[
            pltpu.VMEM((cpt0, CHG), jnp.int32),
            pltpu.VMEM((cpt0, CHG), jnp.int32),
            pltpu.VMEM((4, CHG, 16), jnp.float32),
            pltpu.VMEM((4, CHG, 16), jnp.float32),
            pltpu.VMEM((4, CHG // 8, 128), jnp.float32),
            pltpu.SemaphoreType.DMA,
            pltpu.SemaphoreType.DMA,
            pltpu.SemaphoreType.DMA,
        ],
        compiler_params=pltpu.CompilerParams(use_tc_tiling_on_sc=False),
    )
    def k(c16_hbm, row_hbm, col_hbm, cd_out,
          rowi, coli, crv, ccv, wbv, sem_g, sem_w, sem_i):
        c = lax.axis_index("c")
        s = lax.axis_index("s")

        def start_gather(kd, slot):
            pltpu.async_copy(c16_hbm.at[rowi.at[kd]], crv.at[slot], sem_g)
            pltpu.async_copy(c16_hbm.at[coli.at[kd]], ccv.at[slot], sem_g)

        def wait_gather():
            pltpu.make_async_copy(c16_hbm.at[pl.ds(0, CHG)], crv.at[0], sem_g).wait()
            pltpu.make_async_copy(c16_hbm.at[pl.ds(0, CHG)], ccv.at[0], sem_g).wait()

        def run(start_chunk, cptw):
            base0 = start_chunk * CHG
            pltpu.async_copy(row_hbm.at[pl.ds(start_chunk, cptw)],
                             rowi.at[pl.ds(0, cptw)], sem_i)
            pltpu.async_copy(col_hbm.at[pl.ds(start_chunk, cptw)],
                             coli.at[pl.ds(0, cptw)], sem_i)
            pltpu.make_async_copy(row_hbm.at[pl.ds(0, cptw)],
                                  rowi.at[pl.ds(0, cptw)], sem_i).wait()
            pltpu.make_async_copy(col_hbm.at[pl.ds(0, cptw)],
                                  coli.at[pl.ds(0, cptw)], sem_i).wait()

            def start_wb(kd, slot):
                def sub_body(i, _):
                    for j in range(8):
                        wbv[slot, i, pl.ds(16 * j, 16)] = (
                            crv[slot, 8 * i + j] - ccv[slot, 8 * i + j])
                    return _
                lax.fori_loop(0, CHG // 8, sub_body, None)
                pltpu.async_copy(
                    wbv.at[slot],
                    cd_out.at[pl.ds((base0 + kd * CHG) // 8, CHG // 8)], sem_w)

            def wait_wb():
                pltpu.make_async_copy(cd_out.at[pl.ds(0, CHG // 8)], wbv.at[0],
                                      sem_w).wait()

            _gather_pipeline(cptw, start_gather, wait_gather, start_wb, wait_wb)

        @pl.when(c == 0)
        def _():
            run(s * cpt0, cpt0)

        @pl.when(c == 1)
        def _():
            run(NS * cpt0 + s * cpt1, cpt1)

    return k(coord16, row_g, col_g)


def _radial_perms():
    pas, pbs = [], []
    for dd in range(3):
        pa = np.zeros((128, 128), np.float32)
        pb = np.zeros((128, 128), np.float32)
        for j in range(8):
            for c in range(4):
                for f in range(4):
                    pa[16 * j + 4 * c + dd, 16 * j + 4 * c + f] = 1.0
                    pb[16 * j + 4 * f + dd, 16 * j + 4 * c + f] = 1.0
        pas.append(pa)
        pbs.append(pb)
    return jnp.asarray(np.stack(pas)), jnp.asarray(np.stack(pbs))


def _mod16_mats():
    g = np.zeros((128, 128), np.float32)
    hh = np.zeros((128, 128), np.float32)
    for l in range(128):
        for l2 in range(128):
            if l % 16 == l2 % 16:
                g[l, l2] = 1.0
    for j in range(8):
        for k in range(16):
            hh[16 * j + 15, 16 * j + k] = 1.0
    return jnp.asarray(g), jnp.asarray(hh)


def _radial128(cd, pa_ref, pb_ref):
    r = (cd @ pa_ref[0]) * (cd @ pb_ref[0])
    r += (cd @ pa_ref[1]) * (cd @ pb_ref[1])
    r += (cd @ pa_ref[2]) * (cd @ pb_ref[2])
    return r


def _sumsq(cd128, pa, pb):
    rows = cd128.shape[0]
    blk = 1024

    def body(pa_ref, pb_ref, cd_ref, acc_ref):
        r = _radial128(cd_ref[...], pa_ref, pb_ref)
        r2 = (r * r).reshape(blk // 8, 8, 128)
        part = jnp.sum(r2, axis=0)

        @pl.when(pl.program_id(0) == 0)
        def _():
            acc_ref[...] = jnp.zeros_like(acc_ref)

        acc_ref[...] += part

    return pl.pallas_call(
        body,
        grid=(rows // blk,),
        in_specs=[
            pl.BlockSpec((3, 128, 128), lambda i: (0, 0, 0)),
            pl.BlockSpec((3, 128, 128), lambda i: (0, 0, 0)),
            pl.BlockSpec((blk, 128), lambda i: (i, 0)),
        ],
        out_specs=pl.BlockSpec((8, 128), lambda i: (0, 0)),
        out_shape=jax.ShapeDtypeStruct((8, 128), jnp.float32),
    )(pa, pb, cd128)


def _edge_mlp(sums, ar, bc, cd128, ea, pa, pb, g_mat,
              w_r, w1e, b1, w2, b2, cw1, cb1, cw_pack):
    ep, d = ar.shape
    e = ea.shape[0]
    blk = 6400
    rows = blk // 8

    def body(sums_ref, pa_ref, pb_ref, g_ref, wr_ref, w1e_ref, b1_ref,
             w2_ref, b2_ref, cw1_ref, cb1_ref, cwp_ref,
             ar_ref, bc_ref, cd_ref, ea_ref, ef_ref, tr_ref):
        def bdot(x, w_ref):
            return jnp.dot(x.astype(jnp.bfloat16), w_ref[...],
                           preferred_element_type=jnp.float32)

        tot = jnp.sum(sums_ref[...], axis=0, keepdims=True) @ g_ref[...]
        scale = 1.0 / jnp.maximum(jnp.sqrt(tot), 1e-12)
        cd = cd_ref[...]
        rfn = _radial128(cd, pa_ref, pb_ref) * scale
        contr = bdot(rfn, wr_ref).reshape(blk, d)
        m = jax.nn.silu(ar_ref[...] + bc_ref[...] + contr
                        + bdot(ea_ref[...], w1e_ref) + b1_ref[...])
        ef = jax.nn.silu(bdot(m, w2_ref) + b2_ref[...])
        ef_ref[...] = ef
        cmh = jax.nn.silu(bdot(ef, cw1_ref) + cb1_ref[...])
        cm128 = bdot(cmh.reshape(rows, 8 * d), cwp_ref)
        tr = cd * cm128
        lane = lax.broadcasted_iota(jnp.int32, (rows, 128), 1)
        tr_ref[...] = jnp.where(lane % 16 == 15, 1.0, tr)

    return pl.pallas_call(
        body,
        grid=(e // blk,),
        in_specs=[
            pl.BlockSpec((8, 128), lambda i: (0, 0)),
            pl.BlockSpec((3, 128, 128), lambda i: (0, 0, 0)),
            pl.BlockSpec((3, 128, 128), lambda i: (0, 0, 0)),
            pl.BlockSpec((128, 128), lambda i: (0, 0)),
            pl.BlockSpec((128, 8 * d), lambda i: (0, 0)),
            pl.BlockSpec((16, d), lambda i: (0, 0)),
            pl.BlockSpec((1, d), lambda i: (0, 0)),
            pl.BlockSpec((d, d), lambda i: (0, 0)),
            pl.BlockSpec((1, d), lambda i: (0, 0)),
            pl.BlockSpec((d, d), lambda i: (0, 0)),
            pl.BlockSpec((1, d), lambda i: (0, 0)),
            pl.BlockSpec((8 * d, 128), lambda i: (0, 0)),
            pl.BlockSpec((blk, d), lambda i: (i, 0)),
            pl.BlockSpec((blk, d), lambda i: (i, 0)),
            pl.BlockSpec((rows, 128), lambda i: (i, 0)),
            pl.BlockSpec((blk, 16), lambda i: (i, 0)),
        ],
        out_specs=[
            pl.BlockSpec((blk, d), lambda i: (i, 0)),
            pl.BlockSpec((rows, 128), lambda i: (i, 0)),
        ],
        out_shape=[
            jax.ShapeDtypeStruct((ep, d), jnp.float32),
            jax.ShapeDtypeStruct((ep // 8, 128), jnp.float32),
        ],
    )(sums, pa, pb, g_mat, w_r, w1e, b1, w2, b2, cw1, cb1, cw_pack,
      ar, bc, cd128, ea)


def _sc_scatter(vals, row_s, zeros, acc_rows, w, ep, use_tc_tiling):
    rpt = acc_rows // NS
    cpt = ep // (NW * CH)
    lrows = CH if w == 128 else CH // 8
    mesh = plsc.VectorSubcoreMesh(core_axis_name="c", subcore_axis_name="s")

    scratch = [
        pltpu.VMEM((cpt, CH), jnp.int32),
        pltpu.VMEM((2, lrows, 128), jnp.float32),
        pltpu.VMEM_SHARED((acc_rows, w), jnp.float32),
        pltpu.SemaphoreType.DMA,
        pltpu.SemaphoreType.DMA,
        pltpu.SemaphoreType.DMA,
    ]
    if w == 16:
        scratch.insert(2, pltpu.VMEM((2, CH, 16), jnp.float32))

    @functools.partial(
        pl.kernel,
        out_type=jax.ShapeDtypeStruct((NC, acc_rows, w), jnp.float32),
        mesh=mesh,
        scratch_types=scratch,
        compiler_params=pltpu.CompilerParams(use_tc_tiling_on_sc=use_tc_tiling),
    )
    def k(v_hbm, row_hbm, z_hbm, acc_out, idxi, lv, *rest):
        if w == 16:
            sv, acc_sp, sem_l, sem_s, sem_i = rest
        else:
            acc_sp, sem_l, sem_s, sem_i = rest
        c = lax.axis_index("c")
        s = lax.axis_index("s")
        wid = c * NS + s
        base0 = wid * (cpt * lrows)
        pltpu.async_copy(row_hbm.at[wid], idxi, sem_i)
        pltpu.sync_copy(z_hbm, acc_sp.at[pl.ds(s * rpt, rpt)])
        pltpu.make_async_copy(row_hbm.at[wid], idxi, sem_i).wait()
        plsc.subcore_barrier()

        def start_load(kd, slot):
            pltpu.async_copy(v_hbm.at[pl.ds(base0 + kd * lrows, lrows)],
                             lv.at[slot], sem_l)

        def wait_load():
            pltpu.make_async_copy(v_hbm.at[pl.ds(0, lrows)], lv.at[0],
                                  sem_l).wait()

        def start_scatter(kd, slot):
            if w == 16:
                def rp(i, _):
                    for j in range(8):
                        sv[slot, 8 * i + j] = lv[slot, i, pl.ds(16 * j, 16)]
                    return _
                lax.fori_loop(0, CH // 8, rp, None)
                src = sv.at[slot]
            else:
                src = lv.at[slot]
            pltpu.async_copy(src, acc_sp.at[idxi.at[kd]], sem_s, add=True)

        def wait_scatter():
            pltpu.make_async_copy(v_hbm.at[pl.ds(0, lrows)], lv.at[0],
                                  sem_s).wait()

        start_load(0, 0)
        wait_load()
        start_scatter(0, 0)
        start_load(1, 1)
        wait_load()
        start_scatter(1, 1)
        wait_scatter()
        start_load(2, 0)

        def body(kk, _):
            for b in (0, 1):
                kd = 2 + 2 * kk + b
                wait_load()
                start_scatter(kd, b)
                wait_scatter()
                start_load(jnp.minimum(kd + 1, cpt - 1), 1 - b)
            return _

        lax.fori_loop(0, (cpt - 2) // 2, body, None)
        wait_scatter()
        wait_load()
        plsc.subcore_barrier()
        pltpu.sync_copy(acc_sp.at[pl.ds(s * rpt, rpt)],
                        acc_out.at[c, pl.ds(s * rpt, rpt)])

    return k(vals, row_s, zeros)


def _node_mlp(h_p, naggp, aggp128, c128, h_mat, w1h, w1a, b1, w2, b2):
    np_, d = h_p.shape
    blk = 1024

    def body(h_ref2, w1h_ref, w1a_ref, b1_ref, w2_ref, b2_ref,
             h_ref, np_ref, ap_ref, c16_ref, hout_ref, cout_ref):
        hcur = h_ref[...]
        nagg = np_ref[0] + np_ref[1]
        agg = ap_ref[0] + ap_ref[1]
        t = jax.nn.silu(hcur @ w1h_ref[...] + nagg @ w1a_ref[...] + b1_ref[...])
        hout_ref[...] = t @ w2_ref[...] + b2_ref[...] + hcur
        deg = agg @ h_ref2[...]
        denom = jnp.maximum(deg, 1.0)
        cout_ref[...] = c16_ref[...] + agg / denom

    return pl.pallas_call(
        body,
        grid=(np_ // blk,),
        in_specs=[
            pl.BlockSpec((128, 128), lambda i: (0, 0)),
            pl.BlockSpec((d, d), lambda i: (0, 0)),
            pl.BlockSpec((d, d), lambda i: (0, 0)),
            pl.BlockSpec((1, d), lambda i: (0, 0)),
            pl.BlockSpec((d, d), lambda i: (0, 0)),
            pl.BlockSpec((1, d), lambda i: (0, 0)),
            pl.BlockSpec((blk, d), lambda i: (i, 0)),
            pl.BlockSpec((NC, blk, d), lambda i: (0, i, 0)),
            pl.BlockSpec((NC, blk // 8, 128), lambda i: (0, i, 0)),
            pl.BlockSpec((blk // 8, 128), lambda i: (i, 0)),
        ],
        out_specs=[
            pl.BlockSpec((blk, d), lambda i: (i, 0)),
            pl.BlockSpec((blk // 8, 128), lambda i: (i, 0)),
        ],
        out_shape=[
            jax.ShapeDtypeStruct((np_, d), jnp.float32),
            jax.ShapeDtypeStruct((np_ // 8, 128), jnp.float32),
        ],
    )(h_mat, w1h, w1a, b1, w2, b2, h_p, naggp, aggp128, c128)


def kernel(h, edge_index, coord, edge_attr,
           ew1, eb1, ew2, eb2, nw1, nb1, nw2, nb2, cw1, cb1, cw2):
    n, d = h.shape
    e = edge_index.shape[1]
    c = coord.shape[1]

    row = edge_index[0].astype(jnp.int32)
    col = edge_index[1].astype(jnp.int32)

    ep = -(-e // (NW * CH)) * (NW * CH)
    npad = 1024 * (-(-n // 1024))
    acc_rows = max(npad, NS * 8 * (-(-n // (NS * 8))))

    cpts = ep // (NW * CH)
    row_g = jnp.pad(row, (0, ep - e)).reshape(ep // CHG, CHG)
    col_g = jnp.pad(col, (0, ep - e)).reshape(ep // CHG, CHG)
    row_s = jnp.pad(row, (0, ep - e), constant_values=n).reshape(NW, cpts, CH)
    coord16 = jnp.pad(coord, ((0, 0), (0, 0), (0, 1))).reshape(n, 4 * c)

    w_hc = jnp.concatenate([ew1[:d], ew1[d:2 * d]], axis=1)
    w1x = ew1[2 * d:]
    eye8 = jnp.eye(8, dtype=jnp.float32)
    w_r = jnp.kron(eye8, w1x[:16])
    cw_pack = jnp.kron(eye8, jnp.repeat(cw2, 4, axis=1))
    pa, pb = _radial_perms()
    g_mat, h_mat = _mod16_mats()
    zn = jnp.zeros((acc_rows // NS, d), jnp.float32)
    za = jnp.zeros((acc_rows // NS, 16), jnp.float32)

    a_tab, b_tab = _node_pre(h, w_hc)
    ar, bc = _sc_gather(a_tab, b_tab, row_g, col_g, ep)
    cd128 = _sc_gather_coord(coord16, row_g, col_g, ep)
    sums = _sumsq(cd128, pa, pb)
    bf = jnp.bfloat16
    ef, tr128 = _edge_mlp(sums, ar, bc, cd128, edge_attr, pa, pb, g_mat,
                          w_r.astype(bf), w1x[16:].astype(bf),
                          eb1.reshape(1, d), ew2.astype(bf),
                          eb2.reshape(1, d), cw1.astype(bf),
                          cb1.reshape(1, d), cw_pack.astype(bf))
    naggp = _sc_scatter(ef, row_s, zn, acc_rows, 128, ep, True)
    aggp = _sc_scatter(tr128, row_s, za, acc_rows, 16, ep, False)

    h_p = jnp.pad(h, ((0, npad - n), (0, 0)))
    c128_p = jnp.pad(coord16, ((0, npad - n), (0, 0))).reshape(npad // 8, 128)
    h_out_p, cout128 = _node_mlp(h_p, naggp[:, :npad],
                                 aggp.reshape(NC, acc_rows // 8, 128)[:, :npad // 8],
                                 c128_p, h_mat,
                                 nw1[:d], nw1[d:], nb1.reshape(1, d),
                                 nw2, nb2.reshape(1, d))
    h_out = h_out_p[:n]
    coord_out = cout128.reshape(npad, 16)[:n].reshape(n, c, 4)[:, :, :3]
    return (h_out, coord_out)


# speedup vs baseline: 1.0243x; 1.0243x over previous
import functools

import jax
import jax.numpy as jnp
import numpy as np
from jax import lax
from jax.experimental import pallas as pl
from jax.experimental.pallas import tpu as pltpu
from jax.experimental.pallas import tpu_sc as plsc

NW = 32
NC = 2
NS = 16
CH = 128
CHG = 64
_CORE0_FRAC_AB = 50
_CORE0_FRAC_CD = 50


def _gather_pipeline(cpt, start_gather, wait_gather, start_wb, wait_wb):
    m = cpt // 4
    start_gather(0, 0)
    start_gather(1, 1)
    for k in range(4):
        wait_gather()
        start_wb(k, k % 4)
        if k >= 2:
            wait_wb()
        start_gather(k + 2, (k + 2) % 4)

    def body(kk, carry):
        k0 = 4 * kk
        for b in range(4):
            wait_gather()
            start_wb(k0 + b, b)
            wait_wb()
            start_gather(k0 + b + 2, (b + 2) % 4)
        return carry

    lax.fori_loop(1, m - 1, body, None)
    for k in range(cpt - 4, cpt):
        wait_gather()
        start_wb(k, k % 4)
        wait_wb()
        if k + 2 <= cpt - 1:
            start_gather(k + 2, (k + 2) % 4)
    wait_wb()
    wait_wb()


def _node_pre(h, w_hc):
    n, d = h.shape
    blk = 1000

    def body(h_ref, w_ref, a_ref, b_ref):
        ab = h_ref[...] @ w_ref[...]
        a_ref[...] = ab[:, :d]
        b_ref[...] = ab[:, d:]

    return pl.pallas_call(
        body,
        grid=(n // blk,),
        in_specs=[
            pl.BlockSpec((blk, d), lambda i: (i, 0)),
            pl.BlockSpec((d, 2 * d), lambda i: (0, 0)),
        ],
        out_specs=[
            pl.BlockSpec((blk, d), lambda i: (i, 0)),
            pl.BlockSpec((blk, d), lambda i: (i, 0)),
        ],
        out_shape=[
            jax.ShapeDtypeStruct((n, d), jnp.float32),
            jax.ShapeDtypeStruct((n, d), jnp.float32),
        ],
    )(h, w_hc)


def _sc_gather(a_tab, b_tab, row_g, col_g, ep):
    d = a_tab.shape[1]
    nch = ep // CHG
    per_pair = nch // NS
    cpt0 = _CORE0_FRAC_AB * per_pair // 100 // 4 * 4
    cpt1 = per_pair - cpt0
    mesh = plsc.VectorSubcoreMesh(core_axis_name="c", subcore_axis_name="s")

    @functools.partial(
        pl.kernel,
        out_type=[
            jax.ShapeDtypeStruct((ep, d), jnp.float32),
            jax.ShapeDtypeStruct((ep, d), jnp.float32),
        ],
        mesh=mesh,
        scratch_types=[
            pltpu.VMEM((cpt0, CHG), jnp.int32),
            pltpu.VMEM((cpt0, CHG), jnp.int32),
            pltpu.VMEM((4, CHG, d), jnp.float32),
            pltpu.VMEM((4, CHG, d), jnp.float32),
            pltpu.SemaphoreType.DMA,
            pltpu.SemaphoreType.DMA,
            pltpu.SemaphoreType.DMA,
        ],
    )
    def k(a_hbm, b_hbm, row_hbm, col_hbm,
          ar_out, bc_out,
          rowi, coli, arv, bcv, sem_g, sem_w, sem_i):
        c = lax.axis_index("c")
        s = lax.axis_index("s")

        def start_gather(kd, slot):
            pltpu.async_copy(a_hbm.at[rowi.at[kd]], arv.at[slot], sem_g)
            pltpu.async_copy(b_hbm.at[coli.at[kd]], bcv.at[slot], sem_g)

        def wait_gather():
            pltpu.make_async_copy(a_hbm.at[pl.ds(0, CHG)], arv.at[0], sem_g).wait()
            pltpu.make_async_copy(b_hbm.at[pl.ds(0, CHG)], bcv.at[0], sem_g).wait()

        def run(start_chunk, cptw):
            base0 = start_chunk * CHG
            pltpu.async_copy(row_hbm.at[pl.ds(start_chunk, cptw)],
                             rowi.at[pl.ds(0, cptw)], sem_i)
            pltpu.async_copy(col_hbm.at[pl.ds(start_chunk, cptw)],
                             coli.at[pl.ds(0, cptw)], sem_i)
            pltpu.make_async_copy(row_hbm.at[pl.ds(0, cptw)],
                                  rowi.at[pl.ds(0, cptw)], sem_i).wait()
            pltpu.make_async_copy(col_hbm.at[pl.ds(0, cptw)],
                                  coli.at[pl.ds(0, cptw)], sem_i).wait()

            def start_wb(kd, slot):
                pltpu.async_copy(arv.at[slot],
                                 ar_out.at[pl.ds(base0 + kd * CHG, CHG)], sem_w)
                pltpu.async_copy(bcv.at[slot],
                                 bc_out.at[pl.ds(base0 + kd * CHG, CHG)], sem_w)

            def wait_wb():
                pltpu.make_async_copy(a_hbm.at[pl.ds(0, CHG)], arv.at[0],
                                      sem_w).wait()
                pltpu.make_async_copy(b_hbm.at[pl.ds(0, CHG)], bcv.at[0],
                                      sem_w).wait()

            _gather_pipeline(cptw, start_gather, wait_gather, start_wb, wait_wb)

        @pl.when(c == 0)
        def _():
            run(s * cpt0, cpt0)

        @pl.when(c == 1)
        def _():
            run(NS * cpt0 + s * cpt1, cpt1)

    return k(a_tab, b_tab, row_g, col_g)


def _sc_gather_coord(coord16, row_g, col_g, ep):
    nch = ep // CHG
    per_pair = nch // NS
    cpt0 = _CORE0_FRAC_CD * per_pair // 100 // 4 * 4
    cpt1 = per_pair - cpt0
    mesh = plsc.VectorSubcoreMesh(core_axis_name="c", subcore_axis_name="s")

    @functools.partial(
        pl.kernel,
        out_type=jax.ShapeDtypeStruct((ep // 8, 128), jnp.float32),
        mesh=mesh,
        scratch_types=[
            pltpu.VMEM((cpt0, CHG), jnp.int32),
            pltpu.VMEM((cpt0, CHG), jnp.int32),
            pltpu.VMEM((4, CHG, 16), jnp.float32),
            pltpu.VMEM((4, CHG, 16), jnp.float32),
            pltpu.VMEM((4, CHG // 8, 128), jnp.float32),
            pltpu.SemaphoreType.DMA,
            pltpu.SemaphoreType.DMA,
            pltpu.SemaphoreType.DMA,
        ],
        compiler_params=pltpu.CompilerParams(use_tc_tiling_on_sc=False),
    )
    def k(c16_hbm, row_hbm, col_hbm, cd_out,
          rowi, coli, crv, ccv, wbv, sem_g, sem_w, sem_i):
        c = lax.axis_index("c")
        s = lax.axis_index("s")

        def start_gather(kd, slot):
            pltpu.async_copy(c16_hbm.at[rowi.at[kd]], crv.at[slot], sem_g)
            pltpu.async_copy(c16_hbm.at[coli.at[kd]], ccv.at[slot], sem_g)

        def wait_gather():
            pltpu.make_async_copy(c16_hbm.at[pl.ds(0, CHG)], crv.at[0], sem_g).wait()
            pltpu.make_async_copy(c16_hbm.at[pl.ds(0, CHG)], ccv.at[0], sem_g).wait()

        def run(start_chunk, cptw):
            base0 = start_chunk * CHG
            pltpu.async_copy(row_hbm.at[pl.ds(start_chunk, cptw)],
                             rowi.at[pl.ds(0, cptw)], sem_i)
            pltpu.async_copy(col_hbm.at[pl.ds(start_chunk, cptw)],
                             coli.at[pl.ds(0, cptw)], sem_i)
            pltpu.make_async_copy(row_hbm.at[pl.ds(0, cptw)],
                                  rowi.at[pl.ds(0, cptw)], sem_i).wait()
            pltpu.make_async_copy(col_hbm.at[pl.ds(0, cptw)],
                                  coli.at[pl.ds(0, cptw)], sem_i).wait()

            def start_wb(kd, slot):
                def sub_body(i, _):
                    for j in range(8):
                        wbv[slot, i, pl.ds(16 * j, 16)] = (
                            crv[slot, 8 * i + j] - ccv[slot, 8 * i + j])
                    return _
                lax.fori_loop(0, CHG // 8, sub_body, None)
                pltpu.async_copy(
                    wbv.at[slot],
                    cd_out.at[pl.ds((base0 + kd * CHG) // 8, CHG // 8)], sem_w)

            def wait_wb():
                pltpu.make_async_copy(cd_out.at[pl.ds(0, CHG // 8)], wbv.at[0],
                                      sem_w).wait()

            _gather_pipeline(cptw, start_gather, wait_gather, start_wb, wait_wb)

        @pl.when(c == 0)
        def _():
            run(s * cpt0, cpt0)

        @pl.when(c == 1)
        def _():
            run(NS * cpt0 + s * cpt1, cpt1)

    return k(coord16, row_g, col_g)


def _radial_perms():
    pas, pbs = [], []
    for dd in range(3):
        pa = np.zeros((128, 128), np.float32)
        pb = np.zeros((128, 128), np.float32)
        for j in range(8):
            for c in range(4):
                for f in range(4):
                    pa[16 * j + 4 * c + dd, 16 * j + 4 * c + f] = 1.0
                    pb[16 * j + 4 * f + dd, 16 * j + 4 * c + f] = 1.0
        pas.append(pa)
        pbs.append(pb)
    return jnp.asarray(np.stack(pas)), jnp.asarray(np.stack(pbs))


def _mod16_mats():
    g = np.zeros((128, 128), np.float32)
    hh = np.zeros((128, 128), np.float32)
    for l in range(128):
        for l2 in range(128):
            if l % 16 == l2 % 16:
                g[l, l2] = 1.0
    for j in range(8):
        for k in range(16):
            hh[16 * j + 15, 16 * j + k] = 1.0
    return jnp.asarray(g), jnp.asarray(hh)


def _radial128(cd, pa_ref, pb_ref):
    r = (cd @ pa_ref[0]) * (cd @ pb_ref[0])
    r += (cd @ pa_ref[1]) * (cd @ pb_ref[1])
    r += (cd @ pa_ref[2]) * (cd @ pb_ref[2])
    return r


def _sumsq(cd128, pa, pb):
    rows = cd128.shape[0]
    blk = 1024

    def body(pa_ref, pb_ref, cd_ref, acc_ref):
        r = _radial128(cd_ref[...], pa_ref, pb_ref)
        r2 = (r * r).reshape(blk // 8, 8, 128)
        part = jnp.sum(r2, axis=0)

        @pl.when(pl.program_id(0) == 0)
        def _():
            acc_ref[...] = jnp.zeros_like(acc_ref)

        acc_ref[...] += part

    return pl.pallas_call(
        body,
        grid=(rows // blk,),
        in_specs=[
            pl.BlockSpec((3, 128, 128), lambda i: (0, 0, 0)),
            pl.BlockSpec((3, 128, 128), lambda i: (0, 0, 0)),
            pl.BlockSpec((blk, 128), lambda i: (i, 0)),
        ],
        out_specs=pl.BlockSpec((8, 128), lambda i: (0, 0)),
        out_shape=jax.ShapeDtypeStruct((8, 128), jnp.float32),
    )(pa, pb, cd128)


def _edge_mlp(sums, ar, bc, cd128, ea, pa, pb, g_mat,
              w_r, w1e, b1, w2, b2, cw1, cb1, cw_pack):
    ep, d = ar.shape
    e = ea.shape[0]
    blk = 8000
    rows = blk // 8

    def body(sums_ref, pa_ref, pb_ref, g_ref, wr_ref, w1e_ref, b1_ref,
             w2_ref, b2_ref, cw1_ref, cb1_ref, cwp_ref,
             ar_ref, bc_ref, cd_ref, ea_ref, ef_ref, tr_ref):
        tot = jnp.sum(sums_ref[...], axis=0, keepdims=True) @ g_ref[...]
        scale = 1.0 / jnp.maximum(jnp.sqrt(tot), 1e-12)
        cd = cd_ref[...]
        rfn = _radial128(cd, pa_ref, pb_ref) * scale
        contr = (rfn @ wr_ref[...]).reshape(blk, d)
        m = jax.nn.silu(ar_ref[...] + bc_ref[...] + contr
                        + ea_ref[...] @ w1e_ref[...] + b1_ref[...])
        ef = jax.nn.silu(m @ w2_ref[...] + b2_ref[...])
        ef_ref[...] = ef
        cmh = jax.nn.silu(ef @ cw1_ref[...] + cb1_ref[...])
        cm128 = cmh.reshape(rows, 8 * d) @ cwp_ref[...]
        tr = cd * cm128
        lane = lax.broadcasted_iota(jnp.int32, (rows, 128), 1)
        tr_ref[...] = jnp.where(lane % 16 == 15, 1.0, tr)

    return pl.pallas_call(
        body,
        grid=(e // blk,),
        in_specs=[
            pl.BlockSpec((8, 128), lambda i: (0, 0)),
            pl.BlockSpec((3, 128, 128), lambda i: (0, 0, 0)),
            pl.BlockSpec((3, 128, 128), lambda i: (0, 0, 0)),
            pl.BlockSpec((128, 128), lambda i: (0, 0)),
            pl.BlockSpec((128, 8 * d), lambda i: (0, 0)),
            pl.BlockSpec((16, d), lambda i: (0, 0)),
            pl.BlockSpec((1, d), lambda i: (0, 0)),
            pl.BlockSpec((d, d), lambda i: (0, 0)),
            pl.BlockSpec((1, d), lambda i: (0, 0)),
            pl.BlockSpec((d, d), lambda i: (0, 0)),
            pl.BlockSpec((1, d), lambda i: (0, 0)),
            pl.BlockSpec((8 * d, 128), lambda i: (0, 0)),
            pl.BlockSpec((blk, d), lambda i: (i, 0)),
            pl.BlockSpec((blk, d), lambda i: (i, 0)),
            pl.BlockSpec((rows, 128), lambda i: (i, 0)),
            pl.BlockSpec((blk, 16), lambda i: (i, 0)),
        ],
        out_specs=[
            pl.BlockSpec((blk, d), lambda i: (i, 0)),
            pl.BlockSpec((rows, 128), lambda i: (i, 0)),
        ],
        out_shape=[
            jax.ShapeDtypeStruct((ep, d), jnp.float32),
            jax.ShapeDtypeStruct((ep // 8, 128), jnp.float32),
        ],
    )(sums, pa, pb, g_mat, w_r, w1e, b1, w2, b2, cw1, cb1, cw_pack,
      ar, bc, cd128, ea)


def _sc_scatter(vals, row_s, zeros, acc_rows, w, ep, use_tc_tiling):
    rpt = acc_rows // NS
    cpt = ep // (NW * CH)
    lrows = CH if w == 128 else CH // 8
    mesh = plsc.VectorSubcoreMesh(core_axis_name="c", subcore_axis_name="s")

    scratch = [
        pltpu.VMEM((cpt, CH), jnp.int32),
        pltpu.VMEM((2, lrows, 128), jnp.float32),
        pltpu.VMEM_SHARED((acc_rows, w), jnp.float32),
        pltpu.SemaphoreType.DMA,
        pltpu.SemaphoreType.DMA,
        pltpu.SemaphoreType.DMA,
    ]
    if w == 16:
        scratch.insert(2, pltpu.VMEM((2, CH, 16), jnp.float32))

    @functools.partial(
        pl.kernel,
        out_type=jax.ShapeDtypeStruct((NC, acc_rows, w), jnp.float32),
        mesh=mesh,
        scratch_types=scratch,
        compiler_params=pltpu.CompilerParams(use_tc_tiling_on_sc=use_tc_tiling),
    )
    def k(v_hbm, row_hbm, z_hbm, acc_out, idxi, lv, *rest):
        if w == 16:
            sv, acc_sp, sem_l, sem_s, sem_i = rest
        else:
            acc_sp, sem_l, sem_s, sem_i = rest
        c = lax.axis_index("c")
        s = lax.axis_index("s")
        wid = c * NS + s
        base0 = wid * (cpt * lrows)
        pltpu.async_copy(row_hbm.at[wid], idxi, sem_i)
        pltpu.sync_copy(z_hbm, acc_sp.at[pl.ds(s * rpt, rpt)])
        pltpu.make_async_copy(row_hbm.at[wid], idxi, sem_i).wait()
        plsc.subcore_barrier()

        def start_load(kd, slot):
            pltpu.async_copy(v_hbm.at[pl.ds(base0 + kd * lrows, lrows)],
                             lv.at[slot], sem_l)

        def wait_load():
            pltpu.make_async_copy(v_hbm.at[pl.ds(0, lrows)], lv.at[0],
                                  sem_l).wait()

        def start_scatter(kd, slot):
            if w == 16:
                def rp(i, _):
                    for j in range(8):
                        sv[slot, 8 * i + j] = lv[slot, i, pl.ds(16 * j, 16)]
                    return _
                lax.fori_loop(0, CH // 8, rp, None)
                src = sv.at[slot]
            else:
                src = lv.at[slot]
            pltpu.async_copy(src, acc_sp.at[idxi.at[kd]], sem_s, add=True)

        def wait_scatter():
            pltpu.make_async_copy(v_hbm.at[pl.ds(0, lrows)], lv.at[0],
                                  sem_s).wait()

        start_load(0, 0)
        wait_load()
        start_scatter(0, 0)
        start_load(1, 1)
        wait_load()
        start_scatter(1, 1)
        wait_scatter()
        start_load(2, 0)

        def body(kk, _):
            for b in (0, 1):
                kd = 2 + 2 * kk + b
                wait_load()
                start_scatter(kd, b)
                wait_scatter()
                start_load(jnp.minimum(kd + 1, cpt - 1), 1 - b)
            return _

        lax.fori_loop(0, (cpt - 2) // 2, body, None)
        wait_scatter()
        wait_load()
        plsc.subcore_barrier()
        pltpu.sync_copy(acc_sp.at[pl.ds(s * rpt, rpt)],
                        acc_out.at[c, pl.ds(s * rpt, rpt)])

    return k(vals, row_s, zeros)


def _node_mlp(h_p, naggp, aggp128, c128, h_mat, w1h, w1a, b1, w2, b2):
    np_, d = h_p.shape
    blk = 1024

    def body(h_ref2, w1h_ref, w1a_ref, b1_ref, w2_ref, b2_ref,
             h_ref, np_ref, ap_ref, c16_ref, hout_ref, cout_ref):
        hcur = h_ref[...]
        nagg = np_ref[0] + np_ref[1]
        agg = ap_ref[0] + ap_ref[1]
        t = jax.nn.silu(hcur @ w1h_ref[...] + nagg @ w1a_ref[...] + b1_ref[...])
        hout_ref[...] = t @ w2_ref[...] + b2_ref[...] + hcur
        deg = agg @ h_ref2[...]
        denom = jnp.maximum(deg, 1.0)
        cout_ref[...] = c16_ref[...] + agg / denom

    return pl.pallas_call(
        body,
        grid=(np_ // blk,),
        in_specs=[
            pl.BlockSpec((128, 128), lambda i: (0, 0)),
            pl.BlockSpec((d, d), lambda i: (0, 0)),
            pl.BlockSpec((d, d), lambda i: (0, 0)),
            pl.BlockSpec((1, d), lambda i: (0, 0)),
            pl.BlockSpec((d, d), lambda i: (0, 0)),
            pl.BlockSpec((1, d), lambda i: (0, 0)),
            pl.BlockSpec((blk, d), lambda i: (i, 0)),
            pl.BlockSpec((NC, blk, d), lambda i: (0, i, 0)),
            pl.BlockSpec((NC, blk // 8, 128), lambda i: (0, i, 0)),
            pl.BlockSpec((blk // 8, 128), lambda i: (i, 0)),
        ],
        out_specs=[
            pl.BlockSpec((blk, d), lambda i: (i, 0)),
            pl.BlockSpec((blk // 8, 128), lambda i: (i, 0)),
        ],
        out_shape=[
            jax.ShapeDtypeStruct((np_, d), jnp.float32),
            jax.ShapeDtypeStruct((np_ // 8, 128), jnp.float32),
        ],
    )(h_mat, w1h, w1a, b1, w2, b2, h_p, naggp, aggp128, c128)


def kernel(h, edge_index, coord, edge_attr,
           ew1, eb1, ew2, eb2, nw1, nb1, nw2, nb2, cw1, cb1, cw2):
    n, d = h.shape
    e = edge_index.shape[1]
    c = coord.shape[1]

    row = edge_index[0].astype(jnp.int32)
    col = edge_index[1].astype(jnp.int32)

    ep = -(-e // (NW * CH)) * (NW * CH)
    npad = 1024 * (-(-n // 1024))
    acc_rows = max(npad, NS * 8 * (-(-n // (NS * 8))))

    cpts = ep // (NW * CH)
    row_g = jnp.pad(row, (0, ep - e)).reshape(ep // CHG, CHG)
    col_g = jnp.pad(col, (0, ep - e)).reshape(ep // CHG, CHG)
    row_s = jnp.pad(row, (0, ep - e), constant_values=n).reshape(NW, cpts, CH)
    coord16 = jnp.pad(coord, ((0, 0), (0, 0), (0, 1))).reshape(n, 4 * c)

    w_hc = jnp.concatenate([ew1[:d], ew1[d:2 * d]], axis=1)
    w1x = ew1[2 * d:]
    eye8 = jnp.eye(8, dtype=jnp.float32)
    w_r = jnp.kron(eye8, w1x[:16])
    cw_pack = jnp.kron(eye8, jnp.repeat(cw2, 4, axis=1))
    pa, pb = _radial_perms()
    g_mat, h_mat = _mod16_mats()
    zn = jnp.zeros((acc_rows // NS, d), jnp.float32)
    za = jnp.zeros((acc_rows // NS, 16), jnp.float32)

    a_tab, b_tab = _node_pre(h, w_hc)
    ar, bc = _sc_gather(a_tab, b_tab, row_g, col_g, ep)
    cd128 = _sc_gather_coord(coord16, row_g, col_g, ep)
    sums = _sumsq(cd128, pa, pb)
    ef, tr128 = _edge_mlp(sums, ar, bc, cd128, edge_attr, pa, pb, g_mat,
                          w_r, w1x[16:], eb1.reshape(1, d), ew2,
                          eb2.reshape(1, d), cw1, cb1.reshape(1, d), cw_pack)
    naggp = _sc_scatter(ef, row_s, zn, acc_rows, 128, ep, True)
    aggp = _sc_scatter(tr128, row_s, za, acc_rows, 16, ep, False)

    h_p = jnp.pad(h, ((0, npad - n), (0, 0)))
    c128_p = jnp.pad(coord16, ((0, npad - n), (0, 0))).reshape(npad // 8, 128)
    h_out_p, cout128 = _node_mlp(h_p, naggp[:, :npad],
                                 aggp.reshape(NC, acc_rows // 8, 128)[:, :npad // 8],
                                 c128_p, h_mat,
                                 nw1[:d], nw1[d:], nb1.reshape(1, d),
                                 nw2, nb2.reshape(1, d))
    h_out = h_out_p[:n]
    coord_out = cout128.reshape(npad, 16)[:n].reshape(n, c, 4)[:, :, :3]
    return (h_out, coord_out)
